# Initial kernel scaffold; baseline (speedup 1.0000x reference)
import jax
import jax.numpy as jnp
from jax.experimental import pallas as pl


def kernel(x, edge_index, W1a, b1a, W2a, b2a, W1b, b1b, W2b, b2b):
    raise NotImplementedError("write your pallas kernel here")


# speedup vs baseline: 2.9011x; 2.9011x over previous
import functools

import jax
import jax.numpy as jnp
from jax import lax
from jax.experimental import pallas as pl
from jax.experimental.pallas import tpu as pltpu
import jax.experimental.pallas.tpu_sc as plsc

N = 10000
E = 160000
EPS = 1e-7

NSC = 2
NTILE = 16
NW = NSC * NTILE
BE = 128
NB = (E + NW * BE - 1) // (NW * BE)
E_PAD = NW * NB * BE
STRIPE = 632
NP = NTILE * STRIPE
ZROWS = 64


def _make_segsum(n_sub, d_sub):
    mesh = plsc.VectorSubcoreMesh(core_axis_name="c", subcore_axis_name="s")

    def body(tab, srcs, dsts, out, src_v, dst_v, gbuf, zbuf, acc, sem):
        c = lax.axis_index("c")
        s = lax.axis_index("s")
        wid = c * NTILE + s
        row0 = s * STRIPE

        @pl.loop(0, ZROWS)
        def _zfill(r):
            for i in range(d_sub // 16):
                zbuf[r, pl.ds(i * 16, 16)] = jnp.zeros((16,), jnp.float32)

        def zero_stripe():
            nfull = STRIPE // ZROWS
            for t in range(nfull):
                pltpu.sync_copy(zbuf, acc.at[pl.ds(row0 + t * ZROWS, ZROWS)])
            rem = STRIPE - nfull * ZROWS
            if rem:
                pltpu.sync_copy(zbuf.at[pl.ds(0, rem)],
                                acc.at[pl.ds(row0 + nfull * ZROWS, rem)])

        zero_stripe()
        pltpu.sync_copy(dsts.at[wid], dst_v)
        plsc.subcore_barrier()

        for ck in range(n_sub):
            pltpu.sync_copy(srcs.at[ck].at[wid], src_v)

            @pl.loop(0, NB)
            def _blk(j):
                pltpu.async_copy(tab.at[src_v.at[j]], gbuf, sem).wait()
                pltpu.sync_copy(gbuf, acc.at[dst_v.at[j]], add=True)

            plsc.subcore_barrier()
            pltpu.sync_copy(acc.at[pl.ds(row0, STRIPE)],
                            out.at[c].at[ck].at[pl.ds(row0, STRIPE)])
            if ck < n_sub - 1:
                zero_stripe()
            plsc.subcore_barrier()

    return pl.kernel(
        body,
        out_type=jax.ShapeDtypeStruct((NSC, n_sub, NP, d_sub), jnp.float32),
        mesh=mesh,
        scratch_types=[
            pltpu.VMEM((NB, BE), jnp.int32),
            pltpu.VMEM((NB, BE), jnp.int32),
            pltpu.VMEM((BE, d_sub), jnp.float32),
            pltpu.VMEM((ZROWS, d_sub), jnp.float32),
            pltpu.VMEM_SHARED((NP, d_sub), jnp.float32),
            pltpu.SemaphoreType.DMA,
        ],
        compiler_params=pltpu.CompilerParams(use_tc_tiling_on_sc=False),
    )


_make_segsum = functools.lru_cache(maxsize=None)(_make_segsum)


def _segsum_a(*args):
    return _make_segsum(2, 128)(*args)


def _segsum_b(*args):
    return _make_segsum(10, 80)(*args)


def _tc_stage1(x):
    rb = 1000

    def body(x_ref, t_ref):
        y = jnp.maximum(x_ref[...], 0.0) + EPS
        p = jnp.exp(y)
        t_ref[...] = jnp.concatenate([p * y, p], axis=1)

    return pl.pallas_call(
        body,
        grid=(N // rb,),
        in_specs=[pl.BlockSpec((rb, 128), lambda i: (i, 0))],
        out_specs=pl.BlockSpec((rb, 256), lambda i: (i, 0)),
        out_shape=jax.ShapeDtypeStruct((N, 256), jnp.float32),
    )(x)


def _tc_stage2(x, parts, W1, b1, W2, b2):
    rb = 1000

    def body(x_ref, pa_ref, W1_ref, b1_ref, W2_ref, b2_ref, h_ref, t_ref):
        pa = pa_ref[...]
        num = pa[0, 0] + pa[1, 0]
        den = pa[0, 1] + pa[1, 1]
        aggr = num / (den + 1e-30)
        h0 = x_ref[...] + aggr
        z = jnp.maximum(
            jnp.dot(h0, W1_ref[...], preferred_element_type=jnp.float32)
            + b1_ref[...], 0.0)
        h = jnp.maximum(
            jnp.dot(z, W2_ref[...], preferred_element_type=jnp.float32)
            + b2_ref[...], 0.0)
        h_ref[...] = h
        y = h + EPS
        p = jnp.exp(y)
        t_ref[...] = jnp.concatenate([p * y, p], axis=1)

    return pl.pallas_call(
        body,
        grid=(N // rb,),
        in_specs=[
            pl.BlockSpec((rb, 128), lambda i: (i, 0)),
            pl.BlockSpec((2, 2, rb, 128), lambda i: (0, 0, i, 0)),
            pl.BlockSpec((128, 256), lambda i: (0, 0)),
            pl.BlockSpec((1, 256), lambda i: (0, 0)),
            pl.BlockSpec((256, 400), lambda i: (0, 0)),
            pl.BlockSpec((1, 400), lambda i: (0, 0)),
        ],
        out_specs=[
            pl.BlockSpec((rb, 400), lambda i: (i, 0)),
            pl.BlockSpec((rb, 800), lambda i: (i, 0)),
        ],
        out_shape=[
            jax.ShapeDtypeStruct((N, 400), jnp.float32),
            jax.ShapeDtypeStruct((N, 800), jnp.float32),
        ],
    )(x, parts, W1, b1.reshape(1, -1), W2, b2.reshape(1, -1))


def _tc_stage3(h, parts, W1, b1, W2, b2):
    rb = 400

    def body(h_ref, pa_ref, W1_ref, b1_ref, W2_ref, b2_ref, o_ref):
        pa = pa_ref[...]
        ssum = pa[0] + pa[1]
        full = jnp.concatenate([ssum[k] for k in range(10)], axis=1)
        num = full[:, :400]
        den = full[:, 400:]
        aggr = num / (den + 1e-30)
        g = h_ref[...] + aggr
        z = jnp.maximum(
            jnp.dot(g, W1_ref[...], preferred_element_type=jnp.float32)
            + b1_ref[...], 0.0)
        o_ref[...] = jnp.maximum(
            jnp.dot(z, W2_ref[...], preferred_element_type=jnp.float32)
            + b2_ref[...], 0.0)

    return pl.pallas_call(
        body,
        grid=(N // rb,),
        in_specs=[
            pl.BlockSpec((rb, 400), lambda i: (i, 0)),
            pl.BlockSpec((2, 10, rb, 80), lambda i: (0, 0, i, 0)),
            pl.BlockSpec((400, 800), lambda i: (0, 0)),
            pl.BlockSpec((1, 800), lambda i: (0, 0)),
            pl.BlockSpec((800, 4), lambda i: (0, 0)),
            pl.BlockSpec((1, 4), lambda i: (0, 0)),
        ],
        out_specs=pl.BlockSpec((rb, 4), lambda i: (i, 0)),
        out_shape=jax.ShapeDtypeStruct((N, 4), jnp.float32),
    )(h, parts, W1, b1.reshape(1, -1), W2, b2.reshape(1, -1))


def kernel(x, edge_index, W1a, b1a, W2a, b2a, W1b, b1b, W2b, b2b):
    src = edge_index[0]
    dst = edge_index[1]
    pad = E_PAD - E
    src_p = jnp.concatenate([src, jnp.zeros((pad,), jnp.int32)])
    dst_p = jnp.concatenate([dst, jnp.full((pad,), N, jnp.int32)])
    dsts = dst_p.reshape(NW, NB, BE)
    base = src_p.reshape(1, NW, NB, BE)
    srcs_a = base * 2 + jnp.arange(2, dtype=jnp.int32).reshape(2, 1, 1, 1)
    srcs_b = base * 10 + jnp.arange(10, dtype=jnp.int32).reshape(10, 1, 1, 1)

    t_a = _tc_stage1(x)
    parts_a = _segsum_a(t_a.reshape(N * 2, 128), srcs_a, dsts)
    h, t_b = _tc_stage2(x, parts_a, W1a, b1a, W2a, b2a)
    parts_b = _segsum_b(t_b.reshape(N * 10, 80), srcs_b, dsts)
    return _tc_stage3(h, parts_b, W1b, b1b, W2b, b2b)


# speedup vs baseline: 3.7605x; 1.2962x over previous
import functools

import jax
import jax.numpy as jnp
from jax import lax
from jax.experimental import pallas as pl
from jax.experimental.pallas import tpu as pltpu
import jax.experimental.pallas.tpu_sc as plsc

N = 10000
E = 160000
EPS = 1e-7

NSC = 2
NTILE = 16
NW = NSC * NTILE
EPW = 5120
E_PAD = NW * EPW
STRIPE = 632
NP = NTILE * STRIPE
ZROWS = 32


def _make_segsum(n_sub, d_sub, be, nbuf):
    mesh = plsc.VectorSubcoreMesh(core_axis_name="c", subcore_axis_name="s")
    nb = EPW // be
    ngrp = nb // nbuf

    def body(tab, srcs, dsts, out, *rest):
        src_v, dst_v = rest[0], rest[1]
        gbufs = list(rest[2:2 + nbuf])
        zbuf = rest[2 + nbuf]
        acc = rest[3 + nbuf]
        gsems = list(rest[4 + nbuf:4 + 2 * nbuf])
        ssems = list(rest[4 + 2 * nbuf:4 + 3 * nbuf])
        c = lax.axis_index("c")
        s = lax.axis_index("s")
        wid = c * NTILE + s
        row0 = s * STRIPE

        @pl.loop(0, ZROWS)
        def _zfill(r):
            for i in range(d_sub // 16):
                zbuf[r, pl.ds(i * 16, 16)] = jnp.zeros((16,), jnp.float32)

        def zero_stripe():
            nfull = STRIPE // ZROWS
            for t in range(nfull):
                pltpu.sync_copy(zbuf, acc.at[pl.ds(row0 + t * ZROWS, ZROWS)])
            rem = STRIPE - nfull * ZROWS
            if rem:
                pltpu.sync_copy(zbuf.at[pl.ds(0, rem)],
                                acc.at[pl.ds(row0 + nfull * ZROWS, rem)])

        def fire_gather(b, j):
            pltpu.async_copy(tab.at[src_v.at[j]], gbufs[b], gsems[b])

        def wait_gather(b, j):
            pltpu.make_async_copy(tab.at[src_v.at[j]], gbufs[b],
                                  gsems[b]).wait()

        def fire_scatter(b, j):
            pltpu.async_copy(gbufs[b], acc.at[dst_v.at[j]], ssems[b],
                             add=True)

        def wait_scatter(b, j):
            pltpu.make_async_copy(gbufs[b], acc.at[dst_v.at[j]],
                                  ssems[b]).wait()

        zero_stripe()
        pltpu.sync_copy(dsts.at[wid], dst_v)
        plsc.subcore_barrier()

        for ck in range(n_sub):
            pltpu.sync_copy(srcs.at[ck].at[wid], src_v)
            for b in range(nbuf):
                fire_gather(b, b)

            @pl.loop(0, ngrp - 1)
            def _grp(g):
                j0 = g * nbuf
                for b in range(nbuf):
                    wait_gather(b, j0 + b)
                    fire_scatter(b, j0 + b)
                for b in range(nbuf):
                    wait_scatter(b, j0 + b)
                    fire_gather(b, j0 + nbuf + b)

            j0 = (ngrp - 1) * nbuf
            for b in range(nbuf):
                wait_gather(b, j0 + b)
                fire_scatter(b, j0 + b)
            for b in range(nbuf):
                wait_scatter(b, j0 + b)

            plsc.subcore_barrier()
            pltpu.sync_copy(acc.at[pl.ds(row0, STRIPE)],
                            out.at[c].at[ck].at[pl.ds(row0, STRIPE)])
            if ck < n_sub - 1:
                zero_stripe()
            plsc.subcore_barrier()

    return pl.kernel(
        body,
        out_type=jax.ShapeDtypeStruct((NSC, n_sub, NP, d_sub), jnp.float32),
        mesh=mesh,
        scratch_types=(
            [pltpu.VMEM((EPW // be, be), jnp.int32),
             pltpu.VMEM((EPW // be, be), jnp.int32)]
            + [pltpu.VMEM((be, d_sub), jnp.float32) for _ in range(nbuf)]
            + [pltpu.VMEM((ZROWS, d_sub), jnp.float32),
               pltpu.VMEM_SHARED((NP, d_sub), jnp.float32)]
            + [pltpu.SemaphoreType.DMA for _ in range(2 * nbuf)]
        ),
        compiler_params=pltpu.CompilerParams(use_tc_tiling_on_sc=False),
    )


_make_segsum = functools.lru_cache(maxsize=None)(_make_segsum)


def _segsum_a(*args):
    return _make_segsum(2, 128, 128, 2)(*args)


def _segsum_b(*args):
    return _make_segsum(5, 160, 32, 2)(*args)


def _tc_stage1(x):
    rb = 1000

    def body(x_ref, t_ref):
        y = jnp.maximum(x_ref[...], 0.0) + EPS
        p = jnp.exp(y)
        t_ref[...] = jnp.concatenate([p * y, p], axis=1)

    return pl.pallas_call(
        body,
        grid=(N // rb,),
        in_specs=[pl.BlockSpec((rb, 128), lambda i: (i, 0))],
        out_specs=pl.BlockSpec((rb, 256), lambda i: (i, 0)),
        out_shape=jax.ShapeDtypeStruct((N, 256), jnp.float32),
    )(x)


def _tc_stage2(x, parts, W1, b1, W2, b2):
    rb = 1000

    def body(x_ref, pa_ref, W1_ref, b1_ref, W2_ref, b2_ref, h_ref, t_ref):
        pa = pa_ref[...]
        num = pa[0, 0] + pa[1, 0]
        den = pa[0, 1] + pa[1, 1]
        aggr = num / (den + 1e-30)
        h0 = x_ref[...] + aggr
        z = jnp.maximum(
            jnp.dot(h0, W1_ref[...], preferred_element_type=jnp.float32)
            + b1_ref[...], 0.0)
        h = jnp.maximum(
            jnp.dot(z, W2_ref[...], preferred_element_type=jnp.float32)
            + b2_ref[...], 0.0)
        h_ref[...] = h
        y = h + EPS
        p = jnp.exp(y)
        t_ref[...] = jnp.concatenate([p * y, p], axis=1)

    return pl.pallas_call(
        body,
        grid=(N // rb,),
        in_specs=[
            pl.BlockSpec((rb, 128), lambda i: (i, 0)),
            pl.BlockSpec((2, 2, rb, 128), lambda i: (0, 0, i, 0)),
            pl.BlockSpec((128, 256), lambda i: (0, 0)),
            pl.BlockSpec((1, 256), lambda i: (0, 0)),
            pl.BlockSpec((256, 400), lambda i: (0, 0)),
            pl.BlockSpec((1, 400), lambda i: (0, 0)),
        ],
        out_specs=[
            pl.BlockSpec((rb, 400), lambda i: (i, 0)),
            pl.BlockSpec((rb, 800), lambda i: (i, 0)),
        ],
        out_shape=[
            jax.ShapeDtypeStruct((N, 400), jnp.float32),
            jax.ShapeDtypeStruct((N, 800), jnp.float32),
        ],
    )(x, parts, W1, b1.reshape(1, -1), W2, b2.reshape(1, -1))


def _tc_stage3(h, parts, W1, b1, W2, b2):
    rb = 400

    def body(h_ref, pa_ref, W1_ref, b1_ref, W2_ref, b2_ref, o_ref):
        pa = pa_ref[...]
        ssum = pa[0] + pa[1]
        full = jnp.concatenate([ssum[k] for k in range(5)], axis=1)
        num = full[:, :400]
        den = full[:, 400:]
        aggr = num / (den + 1e-30)
        g = h_ref[...] + aggr
        z = jnp.maximum(
            jnp.dot(g, W1_ref[...], preferred_element_type=jnp.float32)
            + b1_ref[...], 0.0)
        o_ref[...] = jnp.maximum(
            jnp.dot(z, W2_ref[...], preferred_element_type=jnp.float32)
            + b2_ref[...], 0.0)

    return pl.pallas_call(
        body,
        grid=(N // rb,),
        in_specs=[
            pl.BlockSpec((rb, 400), lambda i: (i, 0)),
            pl.BlockSpec((2, 5, rb, 160), lambda i: (0, 0, i, 0)),
            pl.BlockSpec((400, 800), lambda i: (0, 0)),
            pl.BlockSpec((1, 800), lambda i: (0, 0)),
            pl.BlockSpec((800, 4), lambda i: (0, 0)),
            pl.BlockSpec((1, 4), lambda i: (0, 0)),
        ],
        out_specs=pl.BlockSpec((rb, 4), lambda i: (i, 0)),
        out_shape=jax.ShapeDtypeStruct((N, 4), jnp.float32),
    )(h, parts, W1, b1.reshape(1, -1), W2, b2.reshape(1, -1))


def kernel(x, edge_index, W1a, b1a, W2a, b2a, W1b, b1b, W2b, b2b):
    src = edge_index[0]
    dst = edge_index[1]
    pad = E_PAD - E
    src_p = jnp.concatenate([src, jnp.zeros((pad,), jnp.int32)])
    dst_p = jnp.concatenate([dst, jnp.full((pad,), N, jnp.int32)])
    dsts_a = dst_p.reshape(NW, EPW // 128, 128)
    dsts_b = dst_p.reshape(NW, EPW // 32, 32)
    base_a = src_p.reshape(1, NW, EPW // 128, 128)
    base_b = src_p.reshape(1, NW, EPW // 32, 32)
    srcs_a = base_a * 2 + jnp.arange(2, dtype=jnp.int32).reshape(2, 1, 1, 1)
    srcs_b = base_b * 5 + jnp.arange(5, dtype=jnp.int32).reshape(5, 1, 1, 1)

    t_a = _tc_stage1(x)
    parts_a = _segsum_a(t_a.reshape(N * 2, 128), srcs_a, dsts_a)
    h, t_b = _tc_stage2(x, parts_a, W1a, b1a, W2a, b2a)
    parts_b = _segsum_b(t_b.reshape(N * 5, 160), srcs_b, dsts_b)
    return _tc_stage3(h, parts_b, W1b, b1b, W2b, b2b)
